# baseline (speedup 1.0000x reference)
import functools

import jax
import jax.numpy as jnp
from jax import lax
from jax.experimental import pallas as pl
from jax.experimental.pallas import tpu as pltpu
from jax.experimental.pallas import tpu_sc as plsc

VOCAB = 100000
BATCH = 4096
SEQ = 200
DIM = 64
NC = 2
NS = 16
NW = NC * NS
PHASES = DIM // NW
NBUF = 2

_mesh = plsc.VectorSubcoreMesh(core_axis_name="c", subcore_axis_name="s")


@functools.partial(
    pl.kernel,
    out_type=jax.ShapeDtypeStruct((SEQ, DIM, BATCH), jnp.float32),
    mesh=_mesh,
    scratch_types=[
        pltpu.VMEM((VOCAB,), jnp.float32),
        pltpu.VMEM((NBUF, BATCH), jnp.int32),
        pltpu.VMEM((NBUF, BATCH), jnp.float32),
        pltpu.SemaphoreType.DMA((NBUF,)),
        pltpu.SemaphoreType.DMA((NBUF,)),
    ],
    compiler_params=pltpu.CompilerParams(
        use_tc_tiling_on_sc=False, needs_layout_passes=False),
)
def _dmajor_kernel(tT_hbm, xT_hbm, out_hbm, tslice, idx_v, obuf, isem, ssem):
    wid = lax.axis_index("s") * NC + lax.axis_index("c")

    def iload(s, b):
        return pltpu.make_async_copy(xT_hbm.at[s], idx_v.at[b], isem.at[b])

    for phase in range(PHASES):
        d = wid * PHASES + phase
        pltpu.sync_copy(tT_hbm.at[d], tslice)

        def ostore(s, b, d=d):
            return pltpu.make_async_copy(
                obuf.at[b], out_hbm.at[s, d], ssem.at[b])

        for b in range(NBUF):
            iload(b, b).start()

        @pl.loop(0, SEQ, step=NBUF)
        def _seq(s0):
            for b in range(NBUF):
                s = s0 + b
                iload(s, b).wait()

                @pl.when(s >= NBUF)
                def _prev_store_done():
                    ostore(s - NBUF, b).wait()

                @pl.loop(0, BATCH // 16, unroll=8)
                def _gather(k):
                    ii = idx_v[b, pl.ds(k * 16, 16)]
                    obuf[b, pl.ds(k * 16, 16)] = plsc.load_gather(
                        tslice, [ii])

                ostore(s, b).start()

                @pl.when(s + NBUF < SEQ)
                def _next_idx():
                    iload(s + NBUF, b).start()

        for b in range(NBUF):
            ostore(SEQ - NBUF + b, b).wait()


def kernel(x, table):
    outT = _dmajor_kernel(table.T, x.T.astype(jnp.int32))
    return outT.transpose(2, 0, 1)


# speedup vs baseline: 2.0476x; 2.0476x over previous
import functools

import jax
import jax.numpy as jnp
from jax import lax
from jax.experimental import pallas as pl
from jax.experimental.pallas import tpu as pltpu
from jax.experimental.pallas import tpu_sc as plsc

VOCAB = 100000
BATCH = 4096
SEQ = 200
DIM = 64
NC = 2
NS = 16
NW = NC * NS
PHASES = DIM // NW
NBUF = 2

_mesh = plsc.VectorSubcoreMesh(core_axis_name="c", subcore_axis_name="s")


@functools.partial(
    pl.kernel,
    out_type=jax.ShapeDtypeStruct((SEQ, DIM, BATCH), jnp.float32),
    mesh=_mesh,
    scratch_types=[
        pltpu.VMEM((VOCAB,), jnp.float32),
        pltpu.VMEM((NBUF, BATCH), jnp.int32),
        pltpu.VMEM((NBUF, BATCH), jnp.float32),
        pltpu.SemaphoreType.DMA((NBUF,)),
        pltpu.SemaphoreType.DMA((NBUF,)),
    ],
    compiler_params=pltpu.CompilerParams(
        use_tc_tiling_on_sc=False, needs_layout_passes=False),
)
def _dmajor_kernel(tT_hbm, xT_hbm, out_hbm, tslice, idx_v, obuf, isem, ssem):
    wid = lax.axis_index("s") * NC + lax.axis_index("c")

    def iload(s, b):
        return pltpu.make_async_copy(xT_hbm.at[s], idx_v.at[b], isem.at[b])

    for phase in range(PHASES):
        d = wid * PHASES + phase
        pltpu.sync_copy(tT_hbm.at[d], tslice)

        def ostore(s, b, d=d):
            return pltpu.make_async_copy(
                obuf.at[b], out_hbm.at[s, d], ssem.at[b])

        for b in range(NBUF):
            iload(b, b).start()

        @pl.loop(0, SEQ, step=NBUF)
        def _seq(s0):
            for b in range(NBUF):
                s = s0 + b
                iload(s, b).wait()

                @pl.when(s >= NBUF)
                def _prev_store_done():
                    ostore(s - NBUF, b).wait()

                @plsc.parallel_loop(0, BATCH, step=16, unroll=8)
                def _gather(k):
                    ii = idx_v[b, pl.ds(k, 16)]
                    obuf[b, pl.ds(k, 16)] = plsc.load_gather(
                        tslice, [ii])

                ostore(s, b).start()

                @pl.when(s + NBUF < SEQ)
                def _next_idx():
                    iload(s + NBUF, b).start()

        for b in range(NBUF):
            ostore(SEQ - NBUF + b, b).wait()


def kernel(x, table):
    outT = _dmajor_kernel(table.T, x.T.astype(jnp.int32))
    return outT.transpose(2, 0, 1)


# speedup vs baseline: 2.0518x; 1.0020x over previous
import functools

import jax
import jax.numpy as jnp
from jax import lax
from jax.experimental import pallas as pl
from jax.experimental.pallas import tpu as pltpu
from jax.experimental.pallas import tpu_sc as plsc

VOCAB = 100000
BATCH = 4096
SEQ = 200
DIM = 64
NC = 2
NS = 16
NW = NC * NS
PHASES = DIM // NW
NBUF = 2

_mesh = plsc.VectorSubcoreMesh(core_axis_name="c", subcore_axis_name="s")


@functools.partial(
    pl.kernel,
    out_type=jax.ShapeDtypeStruct((SEQ, DIM, BATCH), jnp.float32),
    mesh=_mesh,
    scratch_types=[
        pltpu.VMEM((VOCAB,), jnp.float32),
        pltpu.VMEM((NBUF, BATCH), jnp.int32),
        pltpu.VMEM((NBUF, BATCH), jnp.float32),
        pltpu.SemaphoreType.DMA((NBUF,)),
        pltpu.SemaphoreType.DMA((NBUF,)),
    ],
    compiler_params=pltpu.CompilerParams(
        use_tc_tiling_on_sc=False, needs_layout_passes=False),
)
def _dmajor_kernel(tT_hbm, xT_hbm, out_hbm, tslice, idx_v, obuf, isem, ssem):
    wid = lax.axis_index("s") * NC + lax.axis_index("c")

    def iload(s, b):
        return pltpu.make_async_copy(xT_hbm.at[s], idx_v.at[b], isem.at[b])

    for phase in range(PHASES):
        d = wid * PHASES + phase
        pltpu.sync_copy(tT_hbm.at[d], tslice)

        def ostore(s, b, d=d):
            return pltpu.make_async_copy(
                obuf.at[b], out_hbm.at[s, d], ssem.at[b])

        for b in range(NBUF):
            iload(b, b).start()

        @pl.loop(0, SEQ, step=NBUF)
        def _seq(s0):
            for b in range(NBUF):
                s = s0 + b
                iload(s, b).wait()

                @pl.when(s >= NBUF)
                def _prev_store_done():
                    ostore(s - NBUF, b).wait()

                @plsc.parallel_loop(0, BATCH, step=16, unroll=16)
                def _gather(k):
                    ii = idx_v[b, pl.ds(k, 16)]
                    obuf[b, pl.ds(k, 16)] = plsc.load_gather(
                        tslice, [ii])

                ostore(s, b).start()

                @pl.when(s + NBUF < SEQ)
                def _next_idx():
                    iload(s + NBUF, b).start()

        for b in range(NBUF):
            ostore(SEQ - NBUF + b, b).wait()


def kernel(x, table):
    outT = _dmajor_kernel(table.T, x.T.astype(jnp.int32))
    return outT.transpose(2, 0, 1)


# speedup vs baseline: 2.4739x; 1.2057x over previous
import functools

import jax
import jax.numpy as jnp
from jax import lax
from jax.experimental import pallas as pl
from jax.experimental.pallas import tpu as pltpu
from jax.experimental.pallas import tpu_sc as plsc

VOCAB = 100000
BATCH = 4096
SEQ = 200
DIM = 64
NC = 2
NS = 16
NW = NC * NS
B_BLK = BATCH // NW
PITCH = 129
NBUF = 2

_mesh = plsc.VectorSubcoreMesh(core_axis_name="c", subcore_axis_name="s")


@functools.partial(
    pl.kernel,
    out_type=jax.ShapeDtypeStruct((SEQ, DIM, BATCH), jnp.float32),
    mesh=_mesh,
    scratch_types=[
        pltpu.VMEM((SEQ, B_BLK), jnp.int32),
        pltpu.VMEM((NBUF, B_BLK, DIM), jnp.float32),
        pltpu.VMEM((NBUF, DIM, PITCH), jnp.float32),
        pltpu.SemaphoreType.DMA((NBUF,)),
        pltpu.SemaphoreType.DMA((NBUF,)),
    ],
    compiler_params=pltpu.CompilerParams(
        use_tc_tiling_on_sc=False, needs_layout_passes=False),
)
def _gather_kernel(table_hbm, xT_hbm, out_hbm, idx_v, gbuf, tbuf, gsem, ssem):
    wid = lax.axis_index("s") * NC + lax.axis_index("c")
    b0 = wid * B_BLK
    pltpu.sync_copy(xT_hbm.at[:, pl.ds(b0, B_BLK)], idx_v)

    def gather(s, b):
        return pltpu.make_async_copy(
            table_hbm.at[idx_v.at[s]], gbuf.at[b], gsem.at[b])

    def store(s, b):
        return pltpu.make_async_copy(
            tbuf.at[b, :, pl.ds(0, B_BLK)],
            out_hbm.at[s, :, pl.ds(b0, B_BLK)],
            ssem.at[b])

    iota = lax.iota(jnp.int32, 16)

    for b in range(NBUF):
        gather(b, b).start()

    @pl.loop(0, SEQ, step=NBUF)
    def _seq(s0):
        for b in range(NBUF):
            s = s0 + b
            gather(s, b).wait()

            @pl.when(s >= NBUF)
            def _prev_store_done():
                store(s - NBUF, b).wait()

            @plsc.parallel_loop(0, B_BLK, step=1, unroll=8)
            def _transpose(r):
                rr = jnp.full((16,), r, jnp.int32)
                for q in range(DIM // 16):
                    vals = gbuf[b, r, pl.ds(q * 16, 16)]
                    plsc.store_scatter(tbuf.at[b], [iota + q * 16, rr], vals)

            store(s, b).start()

            @pl.when(s + NBUF < SEQ)
            def _next_gather():
                gather(s + NBUF, b).start()

    for b in range(NBUF):
        store(SEQ - NBUF + b, b).wait()


def kernel(x, table):
    outT = _gather_kernel(table, x.T.astype(jnp.int32))
    return outT.transpose(2, 0, 1)


# speedup vs baseline: 4.5314x; 1.8317x over previous
import functools

import jax
import jax.numpy as jnp
from jax import lax
from jax.experimental import pallas as pl
from jax.experimental.pallas import tpu as pltpu
from jax.experimental.pallas import tpu_sc as plsc

VOCAB = 100000
BATCH = 4096
SEQ = 200
DIM = 64
NC = 2
NS = 16
NW = NC * NS
B_BLK = BATCH // NW
SB = SEQ // 8
DB = DIM // 8
PITCH = 129
NBUF = 2

_mesh = plsc.VectorSubcoreMesh(core_axis_name="c", subcore_axis_name="s")


@functools.partial(
    pl.kernel,
    out_type=jax.ShapeDtypeStruct((SEQ, DB, NW, 8, B_BLK), jnp.float32),
    mesh=_mesh,
    scratch_types=[
        pltpu.VMEM((SB, 8, B_BLK), jnp.int32),
        pltpu.VMEM((NBUF, B_BLK, DIM), jnp.float32),
        pltpu.VMEM((NBUF, DB, 8, PITCH), jnp.float32),
        pltpu.SemaphoreType.DMA((NBUF,)),
        pltpu.SemaphoreType.DMA((NBUF,)),
    ],
    compiler_params=pltpu.CompilerParams(
        use_tc_tiling_on_sc=False, needs_layout_passes=False),
)
def _gather_kernel(table_hbm, x4_hbm, out_hbm, idx_v, gbuf, tbuf, gsem, ssem):
    wid = lax.axis_index("s") * NC + lax.axis_index("c")
    pltpu.sync_copy(x4_hbm.at[:, wid], idx_v)

    def gather(s, b):
        return pltpu.make_async_copy(
            table_hbm.at[idx_v.at[s >> 3, s & 7]], gbuf.at[b], gsem.at[b])

    def store(s, b):
        return pltpu.make_async_copy(
            tbuf.at[b, :, :, pl.ds(0, B_BLK)],
            out_hbm.at[s, :, wid],
            ssem.at[b])

    iota = lax.iota(jnp.int32, 16)
    bands = [(iota + q * 16) >> 3 for q in range(DIM // 16)]
    rows = [(iota + q * 16) & 7 for q in range(DIM // 16)]

    for b in range(NBUF):
        gather(b, b).start()

    @pl.loop(0, SEQ, step=NBUF)
    def _seq(s0):
        for b in range(NBUF):
            s = s0 + b
            gather(s, b).wait()

            @pl.when(s >= NBUF)
            def _prev_store_done():
                store(s - NBUF, b).wait()

            @plsc.parallel_loop(0, B_BLK, step=1, unroll=8)
            def _transpose(r):
                rr = jnp.full((16,), r, jnp.int32)
                for q in range(DIM // 16):
                    vals = gbuf[b, r, pl.ds(q * 16, 16)]
                    plsc.store_scatter(
                        tbuf.at[b], [bands[q], rows[q], rr], vals)

            store(s, b).start()

            @pl.when(s + NBUF < SEQ)
            def _next_gather():
                gather(s + NBUF, b).start()

    for b in range(NBUF):
        store(SEQ - NBUF + b, b).wait()


def kernel(x, table):
    x4 = x.astype(jnp.int32).T.reshape(SB, 8, NW, B_BLK).transpose(0, 2, 1, 3)
    out5 = _gather_kernel(table, x4)
    return out5.transpose(2, 4, 0, 1, 3).reshape(BATCH, SEQ, DIM)
